# Initial kernel scaffold; baseline (speedup 1.0000x reference)
import functools

import jax
import jax.numpy as jnp
from jax.experimental import pallas as pl
from jax.experimental.pallas import tpu as pltpu

_K = 32
_RADIUS = 0.3
_TM = 256
_TMM = 256


def _stage1_body(pts_ref, cen_ref, knn_ref, flat_ref, *, n, k):
    b = pl.program_id(0)
    pts = pts_ref[0]
    cen = cen_ref[0]
    c3 = cen[:, :3]
    b_sq = jnp.sum(pts * pts, axis=0, keepdims=True)
    a_sq = jnp.sum(c3 * c3, axis=1, keepdims=True)
    inner = jnp.dot(c3, pts, preferred_element_type=jnp.float32)
    d2 = jnp.clip(a_sq + b_sq - 2.0 * inner, 0.0, None)
    dist = jnp.sqrt(d2 + 1e-8)
    masked = jnp.where(dist <= _RADIUS, dist, jnp.inf)
    iota = jax.lax.broadcasted_iota(jnp.int32, masked.shape, 1)
    big = jnp.int32(2 ** 30)
    cols = []
    for _ in range(k):
        rowmin = jnp.min(masked, axis=1, keepdims=True)
        cand = jnp.where(masked == rowmin, iota, big)
        idx = jnp.min(cand, axis=1, keepdims=True)
        cols.append(idx)
        masked = jnp.where(iota == idx, jnp.inf, masked)
    knn = jnp.concatenate(cols, axis=1)
    knn_ref[0] = knn
    flat_ref[0] = knn + b * n


def _stage1(points_t, centers16, m_total):
    bb, _, n = points_t.shape
    grid = (bb, m_total // _TM)
    out_shape = [
        jax.ShapeDtypeStruct((bb, m_total, _K), jnp.int32),
        jax.ShapeDtypeStruct((bb, m_total, _K), jnp.int32),
    ]
    return pl.pallas_call(
        functools.partial(_stage1_body, n=n, k=_K),
        grid=grid,
        in_specs=[
            pl.BlockSpec((1, 3, n), lambda b, m: (b, 0, 0)),
            pl.BlockSpec((1, _TM, 16), lambda b, m: (b, m, 0)),
        ],
        out_specs=[
            pl.BlockSpec((1, _TM, _K), lambda b, m: (b, m, 0)),
            pl.BlockSpec((1, _TM, _K), lambda b, m: (b, m, 0)),
        ],
        out_shape=out_shape,
    )(points_t, centers16)


def _stage2_body(nf_ref, cf_ref, w1t_ref, b1_ref, w2t_ref, b2_ref,
                 w3t_ref, b3_ref, out_ref, *, k):
    nf = nf_ref[...]
    cf = cf_ref[...]
    tmm, c = cf.shape
    cfe = jnp.broadcast_to(cf[:, None, :], (tmm, k, c)).reshape(tmm * k, c)
    combo = jnp.concatenate([cfe, nf], axis=1)
    h = jnp.dot(combo, w1t_ref[...], preferred_element_type=jnp.float32,
                precision=jax.lax.Precision.HIGHEST) + b1_ref[...]
    h = jnp.maximum(h, 0.0)
    s = jnp.dot(h, w2t_ref[...], preferred_element_type=jnp.float32,
                precision=jax.lax.Precision.HIGHEST) + b2_ref[...]
    s = jax.nn.sigmoid(s)
    fused = jnp.mean((nf * s).reshape(tmm, k, c), axis=1) + cf
    out = jnp.dot(fused, w3t_ref[...], preferred_element_type=jnp.float32,
                  precision=jax.lax.Precision.HIGHEST) + b3_ref[...]
    out_ref[...] = jnp.maximum(out, 0.0)


def _stage2(nf, cf, w1, b1, w2, b2, w3, b3):
    bm, c = cf.shape
    rows = _TMM * _K
    grid = (bm // _TMM,)
    w1t = w1.T
    w2t = w2.T
    w3t = w3.T
    return pl.pallas_call(
        functools.partial(_stage2_body, k=_K),
        grid=grid,
        in_specs=[
            pl.BlockSpec((rows, c), lambda g: (g, 0)),
            pl.BlockSpec((_TMM, c), lambda g: (g, 0)),
            pl.BlockSpec(w1t.shape, lambda g: (0, 0)),
            pl.BlockSpec((1, c), lambda g: (0, 0)),
            pl.BlockSpec(w2t.shape, lambda g: (0, 0)),
            pl.BlockSpec((1, c), lambda g: (0, 0)),
            pl.BlockSpec(w3t.shape, lambda g: (0, 0)),
            pl.BlockSpec((1, w3.shape[0]), lambda g: (0, 0)),
        ],
        out_specs=pl.BlockSpec((_TMM, w3.shape[0]), lambda g: (g, 0)),
        out_shape=jax.ShapeDtypeStruct((bm, w3.shape[0]), jnp.float32),
    )(nf, cf, w1t, b1.reshape(1, -1), w2t, b2.reshape(1, -1),
      w3t, b3.reshape(1, -1))


def kernel(points, feats, center_idx, W1, b1, W2, b2, W3, b3):
    bb, n, _ = points.shape
    m = center_idx.shape[1]
    c = feats.shape[-1]

    points_t = points.transpose(0, 2, 1)
    points_pad = jnp.pad(points, ((0, 0), (0, 0), (0, 13)))
    feats2 = feats.reshape(bb * n, c)
    offs = (jnp.arange(bb, dtype=jnp.int32) * n)[:, None]
    cidx_flat = (center_idx.astype(jnp.int32) + offs).reshape(-1)

    centers16 = jnp.take(points_pad.reshape(bb * n, 16), cidx_flat, axis=0)
    cfeats = jnp.take(feats2, cidx_flat, axis=0)

    knn, flatk = _stage1(points_t, centers16.reshape(bb, m, 16), m)

    nf = jnp.take(feats2, flatk.reshape(-1), axis=0)

    out = _stage2(nf, cfeats, W1, b1, W2, b2, W3, b3)
    return (out.reshape(bb, m, -1), knn)


# speedup vs baseline: 5.6681x; 5.6681x over previous
import functools

import jax
import jax.numpy as jnp
from jax.experimental import pallas as pl
from jax.experimental.pallas import tpu as pltpu

_K = 32
_RADIUS = 0.3
_TM = 256
_TMM = 256


def _fma_sq(v, t):
    p = v * v
    c = v * 4097.0
    vh = c - (c - v)
    vl = v - vh
    e = ((vh * vh - p) + 2.0 * (vh * vl)) + vl * vl
    s = t + p
    z1 = s - t
    ea = (t - (s - z1)) + (p - z1)
    return s + (ea + e)


def _stage1_body(pts_ref, cen_ref, knn_ref, flat_ref, *, n, k):
    b = pl.program_id(0)
    pts = pts_ref[0]
    cen = cen_ref[0]
    c3 = cen[:, :3]
    b_sq = ((pts[0:1, :] * pts[0:1, :] + pts[1:2, :] * pts[1:2, :])
            + pts[2:3, :] * pts[2:3, :])
    a_sq = _fma_sq(c3[:, 2:3], _fma_sq(c3[:, 1:2],
                                       c3[:, 0:1] * c3[:, 0:1]))
    inner = jnp.dot(c3, pts, preferred_element_type=jnp.float32)
    d2 = jnp.clip(a_sq + b_sq - 2.0 * inner, 0.0, None)
    dist = jnp.sqrt(d2 + 1e-8)
    masked = jnp.where(dist <= _RADIUS, dist, 1e30)
    iota = jax.lax.broadcasted_iota(jnp.int32, masked.shape, 1)
    big = jnp.int32(2 ** 30)
    cols = []
    for _ in range(k):
        rowmin = jnp.min(masked, axis=1, keepdims=True)
        cand = jnp.where(masked == rowmin, iota, big)
        idx = jnp.min(cand, axis=1, keepdims=True)
        cols.append(idx)
        masked = jnp.where(iota == idx, 1e31, masked)
    knn = jnp.concatenate(cols, axis=1)
    knn_ref[0] = knn
    flat_ref[0] = knn + b * n


def _stage1(points_t, centers16, m_total):
    bb, _, n = points_t.shape
    tm = min(_TM, m_total)
    grid = (bb, m_total // tm)
    out_shape = [
        jax.ShapeDtypeStruct((bb, m_total, _K), jnp.int32),
        jax.ShapeDtypeStruct((bb, m_total, _K), jnp.int32),
    ]
    return pl.pallas_call(
        functools.partial(_stage1_body, n=n, k=_K),
        grid=grid,
        in_specs=[
            pl.BlockSpec((1, 3, n), lambda b, m: (b, 0, 0)),
            pl.BlockSpec((1, tm, 16), lambda b, m: (b, m, 0)),
        ],
        out_specs=[
            pl.BlockSpec((1, tm, _K), lambda b, m: (b, m, 0)),
            pl.BlockSpec((1, tm, _K), lambda b, m: (b, m, 0)),
        ],
        out_shape=out_shape,
    )(points_t, centers16)


def _stage2_body(nf_ref, cf_ref, w1t_ref, b1_ref, w2t_ref, b2_ref,
                 w3t_ref, b3_ref, out_ref, *, k):
    nf = nf_ref[...]
    cf = cf_ref[...]
    tmm, c = cf.shape
    cfe = jnp.broadcast_to(cf[:, None, :], (tmm, k, c)).reshape(tmm * k, c)
    combo = jnp.concatenate([cfe, nf], axis=1)
    h = jnp.dot(combo, w1t_ref[...], preferred_element_type=jnp.float32,
                precision=jax.lax.Precision.HIGHEST) + b1_ref[...]
    h = jnp.maximum(h, 0.0)
    s = jnp.dot(h, w2t_ref[...], preferred_element_type=jnp.float32,
                precision=jax.lax.Precision.HIGHEST) + b2_ref[...]
    s = jax.nn.sigmoid(s)
    fused = jnp.mean((nf * s).reshape(tmm, k, c), axis=1) + cf
    out = jnp.dot(fused, w3t_ref[...], preferred_element_type=jnp.float32,
                  precision=jax.lax.Precision.HIGHEST) + b3_ref[...]
    out_ref[...] = jnp.maximum(out, 0.0)


def _stage2(nf, cf, w1, b1, w2, b2, w3, b3):
    bm, c = cf.shape
    tmm = min(_TMM, bm)
    rows = tmm * _K
    grid = (bm // tmm,)
    w1t = w1.T
    w2t = w2.T
    w3t = w3.T
    return pl.pallas_call(
        functools.partial(_stage2_body, k=_K),
        grid=grid,
        in_specs=[
            pl.BlockSpec((rows, c), lambda g: (g, 0)),
            pl.BlockSpec((tmm, c), lambda g: (g, 0)),
            pl.BlockSpec(w1t.shape, lambda g: (0, 0)),
            pl.BlockSpec((1, c), lambda g: (0, 0)),
            pl.BlockSpec(w2t.shape, lambda g: (0, 0)),
            pl.BlockSpec((1, c), lambda g: (0, 0)),
            pl.BlockSpec(w3t.shape, lambda g: (0, 0)),
            pl.BlockSpec((1, w3.shape[0]), lambda g: (0, 0)),
        ],
        out_specs=pl.BlockSpec((tmm, w3.shape[0]), lambda g: (g, 0)),
        out_shape=jax.ShapeDtypeStruct((bm, w3.shape[0]), jnp.float32),
    )(nf, cf, w1t, b1.reshape(1, -1), w2t, b2.reshape(1, -1),
      w3t, b3.reshape(1, -1))


def kernel(points, feats, center_idx, W1, b1, W2, b2, W3, b3):
    bb, n, _ = points.shape
    m = center_idx.shape[1]
    c = feats.shape[-1]

    points_t = points.transpose(0, 2, 1)
    points_pad = jnp.pad(points, ((0, 0), (0, 0), (0, 13)))
    feats2 = feats.reshape(bb * n, c)
    offs = (jnp.arange(bb, dtype=jnp.int32) * n)[:, None]
    cidx_flat = (center_idx.astype(jnp.int32) + offs).reshape(-1)

    centers16 = jnp.take(points_pad.reshape(bb * n, 16), cidx_flat, axis=0)
    cfeats = jnp.take(feats2, cidx_flat, axis=0)

    knn, flatk = _stage1(points_t, centers16.reshape(bb, m, 16), m)

    nf = jnp.take(feats2, flatk.reshape(-1), axis=0)

    out = _stage2(nf, cfeats, W1, b1, W2, b2, W3, b3)
    return (out.reshape(bb, m, -1), knn)


# speedup vs baseline: 5.6693x; 1.0002x over previous
import functools

import jax
import jax.numpy as jnp
from jax.experimental import pallas as pl
from jax.experimental.pallas import tpu as pltpu

_K = 32
_RADIUS = 0.3
_TM = 256
_TMM = 256


def _fma_sq(v, t):
    p = v * v
    c = v * 4097.0
    vh = c - (c - v)
    vl = v - vh
    e = ((vh * vh - p) + 2.0 * (vh * vl)) + vl * vl
    s = t + p
    z1 = s - t
    ea = (t - (s - z1)) + (p - z1)
    return s + (ea + e)


def _stage1_body(pts_ref, cen_ref, knn_ref, flat_ref, *, n, k):
    b = pl.program_id(0)
    pts = pts_ref[0]
    cen = cen_ref[0]
    c3 = cen[:, :3]
    b_sq = ((pts[0:1, :] * pts[0:1, :] + pts[1:2, :] * pts[1:2, :])
            + pts[2:3, :] * pts[2:3, :])
    a_sq = _fma_sq(c3[:, 2:3], _fma_sq(c3[:, 1:2],
                                       c3[:, 0:1] * c3[:, 0:1]))
    inner = jnp.dot(c3, pts, preferred_element_type=jnp.float32)
    d2 = jnp.clip(a_sq + b_sq - 2.0 * inner, 0.0, None)
    dist = jnp.sqrt(d2 + 1e-8)
    masked = jnp.where(dist <= _RADIUS, dist, 1e30)
    iota = jax.lax.broadcasted_iota(jnp.int32, masked.shape, 1)
    big = jnp.int32(2 ** 30)
    cols = []
    idx = None
    for r in range(k):
        if r:
            masked = jnp.where(iota == idx, 1e31, masked)
        rowmin = jnp.min(masked, axis=1, keepdims=True)
        cand = jnp.where(masked == rowmin, iota, big)
        idx = jnp.min(cand, axis=1, keepdims=True)
        cols.append(idx)
    knn = jnp.concatenate(cols, axis=1)
    knn_ref[0] = knn
    flat_ref[0] = knn + b * n


def _stage1(points_t, centers16, m_total):
    bb, _, n = points_t.shape
    tm = min(_TM, m_total)
    grid = (bb, m_total // tm)
    out_shape = [
        jax.ShapeDtypeStruct((bb, m_total, _K), jnp.int32),
        jax.ShapeDtypeStruct((bb, m_total, _K), jnp.int32),
    ]
    return pl.pallas_call(
        functools.partial(_stage1_body, n=n, k=_K),
        grid=grid,
        in_specs=[
            pl.BlockSpec((1, 3, n), lambda b, m: (b, 0, 0)),
            pl.BlockSpec((1, tm, 16), lambda b, m: (b, m, 0)),
        ],
        out_specs=[
            pl.BlockSpec((1, tm, _K), lambda b, m: (b, m, 0)),
            pl.BlockSpec((1, tm, _K), lambda b, m: (b, m, 0)),
        ],
        out_shape=out_shape,
    )(points_t, centers16)


def _stage2_body(nf_ref, cf_ref, w1t_ref, b1_ref, w2t_ref, b2_ref,
                 w3t_ref, b3_ref, out_ref, *, k):
    nf = nf_ref[...]
    cf = cf_ref[...]
    tmm, c = cf.shape
    cfe = jnp.broadcast_to(cf[:, None, :], (tmm, k, c)).reshape(tmm * k, c)
    combo = jnp.concatenate([cfe, nf], axis=1)
    h = jnp.dot(combo, w1t_ref[...], preferred_element_type=jnp.float32,
                precision=jax.lax.Precision.HIGHEST) + b1_ref[...]
    h = jnp.maximum(h, 0.0)
    s = jnp.dot(h, w2t_ref[...], preferred_element_type=jnp.float32,
                precision=jax.lax.Precision.HIGHEST) + b2_ref[...]
    s = jax.nn.sigmoid(s)
    fused = jnp.mean((nf * s).reshape(tmm, k, c), axis=1) + cf
    out = jnp.dot(fused, w3t_ref[...], preferred_element_type=jnp.float32,
                  precision=jax.lax.Precision.HIGHEST) + b3_ref[...]
    out_ref[...] = jnp.maximum(out, 0.0)


def _stage2(nf, cf, w1, b1, w2, b2, w3, b3):
    bm, c = cf.shape
    tmm = min(_TMM, bm)
    rows = tmm * _K
    grid = (bm // tmm,)
    w1t = w1.T
    w2t = w2.T
    w3t = w3.T
    return pl.pallas_call(
        functools.partial(_stage2_body, k=_K),
        grid=grid,
        in_specs=[
            pl.BlockSpec((rows, c), lambda g: (g, 0)),
            pl.BlockSpec((tmm, c), lambda g: (g, 0)),
            pl.BlockSpec(w1t.shape, lambda g: (0, 0)),
            pl.BlockSpec((1, c), lambda g: (0, 0)),
            pl.BlockSpec(w2t.shape, lambda g: (0, 0)),
            pl.BlockSpec((1, c), lambda g: (0, 0)),
            pl.BlockSpec(w3t.shape, lambda g: (0, 0)),
            pl.BlockSpec((1, w3.shape[0]), lambda g: (0, 0)),
        ],
        out_specs=pl.BlockSpec((tmm, w3.shape[0]), lambda g: (g, 0)),
        out_shape=jax.ShapeDtypeStruct((bm, w3.shape[0]), jnp.float32),
    )(nf, cf, w1t, b1.reshape(1, -1), w2t, b2.reshape(1, -1),
      w3t, b3.reshape(1, -1))


def kernel(points, feats, center_idx, W1, b1, W2, b2, W3, b3):
    bb, n, _ = points.shape
    m = center_idx.shape[1]
    c = feats.shape[-1]

    points_t = points.transpose(0, 2, 1)
    points_pad = jnp.pad(points, ((0, 0), (0, 0), (0, 13)))
    feats2 = feats.reshape(bb * n, c)
    offs = (jnp.arange(bb, dtype=jnp.int32) * n)[:, None]
    cidx_flat = (center_idx.astype(jnp.int32) + offs).reshape(-1)

    centers16 = jnp.take(points_pad.reshape(bb * n, 16), cidx_flat, axis=0)
    cfeats = jnp.take(feats2, cidx_flat, axis=0)

    knn, flatk = _stage1(points_t, centers16.reshape(bb, m, 16), m)

    nf = jnp.take(feats2, flatk.reshape(-1), axis=0)

    out = _stage2(nf, cfeats, W1, b1, W2, b2, W3, b3)
    return (out.reshape(bb, m, -1), knn)
